# Initial kernel scaffold; baseline (speedup 1.0000x reference)
import jax
import jax.numpy as jnp
from jax.experimental import pallas as pl


def kernel(x, table, conv_w, conv_b):
    raise NotImplementedError("write your pallas kernel here")


# speedup vs baseline: 71.2579x; 71.2579x over previous
import functools

import jax
import jax.numpy as jnp
from jax import lax
from jax.experimental import pallas as pl
from jax.experimental.pallas import tpu as pltpu
from jax.experimental.pallas import tpu_sc as plsc

N_BINS = 256
D = 32
B = 16384
S = 200
SP = S + 4
K = 5

NC, NS = 2, 16
NW = NC * NS
ROWS_W = B // NW
R = 4
CHUNKS = ROWS_W // R
PCH = R * S
IV = PCH // 16

G = S // 8
BR = 128


def _sc_gather(x_flat, binsext, table, zeros2):
    mesh = plsc.VectorSubcoreMesh(core_axis_name="c", subcore_axis_name="s")

    @functools.partial(
        pl.kernel,
        mesh=mesh,
        compiler_params=pltpu.CompilerParams(
            needs_layout_passes=False, use_tc_tiling_on_sc=False),
        out_type=jax.ShapeDtypeStruct((B, SP, D), jnp.float32),
        scratch_types=[
            pltpu.VMEM((PCH,), jnp.float32),
            pltpu.VMEM((PCH,), jnp.int32),
            pltpu.VMEM((R, SP, D), jnp.float32),
            pltpu.VMEM((N_BINS,), jnp.float32),
            pltpu.SemaphoreType.DMA,
        ],
    )
    def k(x_hbm, be_hbm, tab_hbm, z_hbm, out_hbm, xb, idxb, stag, be, sem):
        wid = lax.axis_index("s") * NC + lax.axis_index("c")
        pltpu.sync_copy(be_hbm, be)

        for r in range(R):
            pltpu.sync_copy(z_hbm, stag.at[r, pl.ds(0, 2)])
            pltpu.sync_copy(z_hbm, stag.at[r, pl.ds(SP - 2, 2)])

        def chunk_body(ci, carry):
            base = wid * (ROWS_W * S) + ci * PCH
            pltpu.sync_copy(x_hbm.at[pl.ds(base, PCH)], xb)

            def iv_body(i, c2):
                off = pl.multiple_of(i * 16, 16)
                y = xb[pl.ds(off, 16)]
                t = y * jnp.float32(N_BINS - 2)
                c = t.astype(jnp.int32)
                c = c + jnp.where(c.astype(jnp.float32) < t, 1, 0)
                c = jnp.clip(c, 0, N_BINS - 2)
                lo = plsc.load_gather(be, [c])
                hi = plsc.load_gather(be, [c + 1])
                c = c + jnp.where(y > hi, 1, 0) - jnp.where(y <= lo, 1, 0)
                idxb[pl.ds(off, 16)] = jnp.clip(c, 0, N_BINS - 1)
                return c2

            lax.fori_loop(0, IV, iv_body, 0)

            for r in range(R):
                pltpu.async_copy(
                    tab_hbm.at[idxb.at[pl.ds(r * S, S)]],
                    stag.at[r, pl.ds(2, S)],
                    sem,
                ).wait()

            row0 = wid * ROWS_W + ci * R
            pltpu.sync_copy(stag, out_hbm.at[pl.ds(row0, R)])
            return carry

        lax.fori_loop(0, CHUNKS, chunk_body, 0)

    return k(x_flat, binsext, table, zeros2)


def _tc_conv_body(e_ref, w_ref, b_ref, o_ref):
    w = w_ref[...]
    bb = b_ref[...]
    for g in range(G):
        win = e_ref[:, 256 * g: 256 * g + 384]
        acc = lax.dot_general(
            win.astype(jnp.bfloat16), w,
            (((1,), (0,)), ((), ())),
            preferred_element_type=jnp.float32,
        )
        o_ref[:, 256 * g: 256 * g + 256] = jnp.tanh(acc + bb)


def _tc_conv(emb_flat, wbig, bias):
    return pl.pallas_call(
        _tc_conv_body,
        grid=(B // BR,),
        in_specs=[
            pl.BlockSpec((BR, SP * D), lambda i: (i, 0)),
            pl.BlockSpec((12 * D, 8 * D), lambda i: (0, 0)),
            pl.BlockSpec((1, 8 * D), lambda i: (0, 0)),
        ],
        out_specs=pl.BlockSpec((BR, S * D), lambda i: (i, 0)),
        out_shape=jax.ShapeDtypeStruct((B, S * D), jnp.float32),
    )(emb_flat, wbig, bias)


def _fold_weights(conv_w):
    wb = jnp.zeros((12, D, 8, D), dtype=jnp.float32)
    for r in range(8):
        for t in range(K):
            wb = wb.at[r + t, :, r, :].set(conv_w[:, :, t].T)
    return wb.reshape(12 * D, 8 * D).astype(jnp.bfloat16)


def kernel(x, table, conv_w, conv_b):
    bins = jnp.linspace(0.0, 1.0, N_BINS - 1)
    binsext = jnp.concatenate(
        [jnp.full((1,), -1e30, jnp.float32), bins.astype(jnp.float32)])
    wbig = _fold_weights(conv_w)
    bias = jnp.tile(conv_b, 8).reshape(1, 8 * D)

    zeros2 = jnp.zeros((2, D), jnp.float32)
    emb_pad = _sc_gather(x.reshape(B * S), binsext, table, zeros2)
    out = _tc_conv(emb_pad.reshape(B, SP * D), wbig, bias)
    return out.reshape(B, S, D)


# speedup vs baseline: 123.3281x; 1.7307x over previous
import functools

import jax
import jax.numpy as jnp
from jax import lax
from jax.experimental import pallas as pl
from jax.experimental.pallas import tpu as pltpu
from jax.experimental.pallas import tpu_sc as plsc

N_BINS = 256
D = 32
B = 16384
S = 200
SP = S + 4
K = 5

NC, NS = 2, 16
NW = NC * NS
ROWS_W = B // NW
R = 4
CHUNKS = ROWS_W // R
PCH = R * S
IV = PCH // 16

G = S // 8
BR = 128


def _sc_gather(x_flat, binsext, table, zeros2):
    mesh = plsc.VectorSubcoreMesh(core_axis_name="c", subcore_axis_name="s")

    @functools.partial(
        pl.kernel,
        mesh=mesh,
        compiler_params=pltpu.CompilerParams(
            needs_layout_passes=False, use_tc_tiling_on_sc=False),
        out_type=jax.ShapeDtypeStruct((B * SP, D), jnp.float32),
        scratch_types=[
            pltpu.VMEM((PCH,), jnp.float32),
            pltpu.VMEM((PCH,), jnp.int32),
            pltpu.VMEM((R, SP, D), jnp.float32),
            pltpu.VMEM((N_BINS,), jnp.float32),
            pltpu.SemaphoreType.DMA,
        ],
    )
    def k(x_hbm, be_hbm, tab_hbm, z_hbm, out_hbm, xb, idxb, stag, be, sem):
        wid = lax.axis_index("s") * NC + lax.axis_index("c")
        pltpu.sync_copy(be_hbm, be)

        for r in range(R):
            pltpu.sync_copy(z_hbm, stag.at[r, pl.ds(0, 2)])
            pltpu.sync_copy(z_hbm, stag.at[r, pl.ds(SP - 2, 2)])

        def chunk_body(ci, carry):
            base = wid * (ROWS_W * S) + ci * PCH
            pltpu.sync_copy(x_hbm.at[pl.ds(base, PCH)], xb)

            def iv_body(i, c2):
                off = pl.multiple_of(i * 16, 16)
                y = xb[pl.ds(off, 16)]
                t = y * jnp.float32(N_BINS - 2)
                c = t.astype(jnp.int32)
                c = c + jnp.where(c.astype(jnp.float32) < t, 1, 0)
                c = jnp.clip(c, 0, N_BINS - 2)
                lo = plsc.load_gather(be, [c])
                hi = plsc.load_gather(be, [c + 1])
                c = c + jnp.where(y > hi, 1, 0) - jnp.where(y <= lo, 1, 0)
                idxb[pl.ds(off, 16)] = jnp.clip(c, 0, N_BINS - 1)
                return c2

            lax.fori_loop(0, IV, iv_body, 0)

            for r in range(R):
                pltpu.async_copy(
                    tab_hbm.at[idxb.at[pl.ds(r * S, S)]],
                    stag.at[r, pl.ds(2, S)],
                    sem,
                ).wait()

            row0 = wid * ROWS_W + ci * R
            for r in range(R):
                pltpu.sync_copy(
                    stag.at[r], out_hbm.at[pl.ds((row0 + r) * SP, SP)])
            return carry

        lax.fori_loop(0, CHUNKS, chunk_body, 0)

    return k(x_flat, binsext, table, zeros2)


def _tc_conv_body(e_ref, w_ref, b_ref, o_ref):
    w = w_ref[...]
    bb = b_ref[...]
    for g in range(G):
        win = e_ref[:, 256 * g: 256 * g + 384]
        acc = lax.dot_general(
            win.astype(jnp.bfloat16), w,
            (((1,), (0,)), ((), ())),
            preferred_element_type=jnp.float32,
        )
        o_ref[:, 256 * g: 256 * g + 256] = jnp.tanh(acc + bb)


def _tc_conv(emb_flat, wbig, bias):
    return pl.pallas_call(
        _tc_conv_body,
        grid=(B // BR,),
        in_specs=[
            pl.BlockSpec((BR, SP * D), lambda i: (i, 0)),
            pl.BlockSpec((12 * D, 8 * D), lambda i: (0, 0)),
            pl.BlockSpec((1, 8 * D), lambda i: (0, 0)),
        ],
        out_specs=pl.BlockSpec((BR, S * D), lambda i: (i, 0)),
        out_shape=jax.ShapeDtypeStruct((B, S * D), jnp.float32),
    )(emb_flat, wbig, bias)


def _fold_weights(conv_w):
    wb = jnp.zeros((12, D, 8, D), dtype=jnp.float32)
    for r in range(8):
        for t in range(K):
            wb = wb.at[r + t, :, r, :].set(conv_w[:, :, t].T)
    return wb.reshape(12 * D, 8 * D).astype(jnp.bfloat16)


def kernel(x, table, conv_w, conv_b):
    bins = jnp.linspace(0.0, 1.0, N_BINS - 1)
    binsext = jnp.concatenate(
        [jnp.full((1,), -1e30, jnp.float32), bins.astype(jnp.float32)])
    wbig = _fold_weights(conv_w)
    bias = jnp.tile(conv_b, 8).reshape(1, 8 * D)

    zeros2 = jnp.zeros((2, D), jnp.float32)
    emb_pad = _sc_gather(x.reshape(B * S), binsext, table, zeros2)
    out = _tc_conv(emb_pad.reshape(B, SP * D), wbig, bias)
    return out.reshape(B, S, D)


# speedup vs baseline: 133.7179x; 1.0842x over previous
import functools

import jax
import jax.numpy as jnp
from jax import lax
from jax.experimental import pallas as pl
from jax.experimental.pallas import tpu as pltpu
from jax.experimental.pallas import tpu_sc as plsc

N_BINS = 256
D = 32
B = 16384
S = 200
SP = S + 4
K = 5

NC, NS = 2, 16
NW = NC * NS
ROWS_W = B // NW
R = 4
CHUNKS = ROWS_W // R
PCH = R * S
IV = PCH // 16

G = S // 8
BR = 128


def _sc_gather(x_flat, binsext, table, zeros2):
    mesh = plsc.VectorSubcoreMesh(core_axis_name="c", subcore_axis_name="s")

    @functools.partial(
        pl.kernel,
        mesh=mesh,
        compiler_params=pltpu.CompilerParams(
            needs_layout_passes=False, use_tc_tiling_on_sc=False),
        out_type=jax.ShapeDtypeStruct((B * SP, D), jnp.float32),
        scratch_types=[
            pltpu.VMEM((2, PCH), jnp.float32),
            pltpu.VMEM((2, PCH), jnp.int32),
            pltpu.VMEM((2, R, SP, D), jnp.float32),
            pltpu.VMEM((N_BINS,), jnp.float32),
            pltpu.SemaphoreType.DMA,
            pltpu.SemaphoreType.DMA,
            pltpu.SemaphoreType.DMA,
            pltpu.SemaphoreType.DMA,
            pltpu.SemaphoreType.DMA,
            pltpu.SemaphoreType.DMA,
        ],
    )
    def k(x_hbm, be_hbm, tab_hbm, z_hbm, out_hbm, xb, idxb, stag, be,
          sx0, sx1, sg0, sg1, so0, so1):
        wid = lax.axis_index("s") * NC + lax.axis_index("c")
        sx = (sx0, sx1)
        sg = (sg0, sg1)
        so = (so0, so1)
        pltpu.sync_copy(be_hbm, be)

        for p in range(2):
            for r in range(R):
                pltpu.sync_copy(z_hbm, stag.at[p, r, pl.ds(0, 2)])
                pltpu.sync_copy(z_hbm, stag.at[p, r, pl.ds(SP - 2, 2)])

        def xbase(ci):
            return wid * (ROWS_W * S) + ci * PCH

        def fire_x(ci, p):
            pltpu.async_copy(
                x_hbm.at[pl.ds(xbase(ci), PCH)], xb.at[p], sx[p])

        def wait_x(p):
            pltpu.make_async_copy(
                x_hbm.at[pl.ds(0, PCH)], xb.at[p], sx[p]).wait()

        def fire_gathers(ci, p):
            for r in range(R):
                pltpu.async_copy(
                    tab_hbm.at[idxb.at[p, pl.ds(r * S, S)]],
                    stag.at[p, r, pl.ds(2, S)],
                    sg[p],
                )

        def wait_gathers(p):
            for r in range(R):
                pltpu.make_async_copy(
                    tab_hbm.at[idxb.at[p, pl.ds(r * S, S)]],
                    stag.at[p, r, pl.ds(2, S)],
                    sg[p],
                ).wait()

        def fire_out(ci, p):
            row0 = wid * ROWS_W + ci * R
            for r in range(R):
                pltpu.async_copy(
                    stag.at[p, r], out_hbm.at[pl.ds((row0 + r) * SP, SP)],
                    so[p])

        def wait_out(p):
            for r in range(R):
                pltpu.make_async_copy(
                    stag.at[p, r], out_hbm.at[pl.ds(0, SP)], so[p]).wait()

        def idx_compute(p):
            def iv_body(i, c2):
                off = pl.multiple_of(i * 16, 16)
                y = xb[p, pl.ds(off, 16)]
                t = y * jnp.float32(N_BINS - 2)
                c = t.astype(jnp.int32)
                c = c + jnp.where(c.astype(jnp.float32) < t, 1, 0)
                c = jnp.clip(c, 0, N_BINS - 2)
                lo = plsc.load_gather(be, [c])
                hi = plsc.load_gather(be, [c + 1])
                c = c + jnp.where(y > hi, 1, 0) - jnp.where(y <= lo, 1, 0)
                idxb[p, pl.ds(off, 16)] = jnp.clip(c, 0, N_BINS - 1)
                return c2

            lax.fori_loop(0, IV, iv_body, 0)

        fire_x(0, 0)
        fire_x(1, 1)

        def chunk_body(ci2, carry):
            for p in range(2):
                ci = 2 * ci2 + p
                wait_x(p)
                idx_compute(p)

                @pl.when(ci2 >= 1)
                def _():
                    wait_out(p)

                fire_gathers(ci, p)

                @pl.when(ci2 < CHUNKS // 2 - 1)
                def _():
                    fire_x(ci + 2, p)

                if p == 0:
                    @pl.when(ci2 >= 1)
                    def _():
                        wait_gathers(1)
                        fire_out(2 * ci2 - 1, 1)
                else:
                    wait_gathers(0)
                    fire_out(2 * ci2, 0)
            return carry

        lax.fori_loop(0, CHUNKS // 2, chunk_body, 0)

        wait_gathers(1)
        fire_out(CHUNKS - 1, 1)
        wait_out(0)
        wait_out(1)

    return k(x_flat, binsext, table, zeros2)


def _tc_conv_body(e_ref, w_ref, b_ref, o_ref):
    w = w_ref[...]
    bb = b_ref[...]
    for g in range(G):
        win = e_ref[:, 256 * g: 256 * g + 384]
        acc = lax.dot_general(
            win.astype(jnp.bfloat16), w,
            (((1,), (0,)), ((), ())),
            preferred_element_type=jnp.float32,
        )
        o_ref[:, 256 * g: 256 * g + 256] = jnp.tanh(acc + bb)


def _tc_conv(emb_flat, wbig, bias):
    return pl.pallas_call(
        _tc_conv_body,
        grid=(B // BR,),
        in_specs=[
            pl.BlockSpec((BR, SP * D), lambda i: (i, 0)),
            pl.BlockSpec((12 * D, 8 * D), lambda i: (0, 0)),
            pl.BlockSpec((1, 8 * D), lambda i: (0, 0)),
        ],
        out_specs=pl.BlockSpec((BR, S * D), lambda i: (i, 0)),
        out_shape=jax.ShapeDtypeStruct((B, S * D), jnp.float32),
    )(emb_flat, wbig, bias)


def _fold_weights(conv_w):
    wb = jnp.zeros((12, D, 8, D), dtype=jnp.float32)
    for r in range(8):
        for t in range(K):
            wb = wb.at[r + t, :, r, :].set(conv_w[:, :, t].T)
    return wb.reshape(12 * D, 8 * D).astype(jnp.bfloat16)


def kernel(x, table, conv_w, conv_b):
    bins = jnp.linspace(0.0, 1.0, N_BINS - 1)
    binsext = jnp.concatenate(
        [jnp.full((1,), -1e30, jnp.float32), bins.astype(jnp.float32)])
    wbig = _fold_weights(conv_w)
    bias = jnp.tile(conv_b, 8).reshape(1, 8 * D)

    zeros2 = jnp.zeros((2, D), jnp.float32)
    emb_pad = _sc_gather(x.reshape(B * S), binsext, table, zeros2)
    out = _tc_conv(emb_pad.reshape(B, SP * D), wbig, bias)
    return out.reshape(B, S, D)
